# baseline (speedup 1.0000x reference)
import functools

import jax
import jax.numpy as jnp
from jax import lax
from jax.experimental import pallas as pl
from jax.experimental.pallas import tpu as pltpu
from jax.experimental.pallas import tpu_sc as plsc

_ROWS = 128
_VOCAB = 100000
_LANES = 16
_NC = 2
_NS = 16
_NW = _NC * _NS
_SC_ROWS = 64
_ROWS_PER_W = _SC_ROWS // _NW
_TC_ROWS = _ROWS - _SC_ROWS

_CSIZE = 51200
_CHUNKS = ((0, _CSIZE), (99840 - _CSIZE, _CSIZE))
_CPR = len(_CHUNKS)
_TAIL0 = 99840
_TAILN = 256
_NSTREAM = 4
_SEGV = _CSIZE // _LANES // _NSTREAM
_IMAX = 2**31 - 1
_UNROLL = 8


def _merge(m_a, bi_a, m_b, bi_b):
    gt = m_b > m_a
    return jnp.where(gt, m_b, m_a), jnp.where(gt, bi_b, bi_a)


def _make_sc_argmax():
    mesh = plsc.VectorSubcoreMesh(core_axis_name="c", subcore_axis_name="s")

    @functools.partial(
        pl.kernel,
        mesh=mesh,
        out_type=jax.ShapeDtypeStruct((_NW, _LANES), jnp.int32),
        compiler_params=pltpu.CompilerParams(needs_layout_passes=False),
        scratch_types=[
            pltpu.VMEM((_CSIZE,), jnp.float32),
            pltpu.VMEM((_CSIZE,), jnp.float32),
            pltpu.VMEM((_ROWS_PER_W * _TAILN,), jnp.float32),
            pltpu.VMEM((_LANES,), jnp.int32),
            pltpu.SemaphoreType.DMA,
            pltpu.SemaphoreType.DMA,
            pltpu.SemaphoreType.DMA,
        ],
    )
    def body(logits_hbm, tail_hbm, out_hbm, buf0, buf1, tailbuf, res_ref,
             sem0, sem1, sem2):
        wid = lax.axis_index("s") * _NC + lax.axis_index("c")
        lanes = lax.iota(jnp.int32, _LANES)
        res_ref[...] = jnp.zeros((_LANES,), jnp.int32)

        bufs = (buf0, buf1)
        sems = (sem0, sem1)
        nt = _ROWS_PER_W * _CPR

        def start(t):
            row = wid * _ROWS_PER_W + t // _CPR
            off, size = _CHUNKS[t % _CPR]
            return pltpu.async_copy(
                logits_hbm.at[row].at[pl.ds(off, size)],
                bufs[t % 2], sems[t % 2])

        def scan_chunk(buf, chunk_off, m, bi):
            def step(i, carry):
                ms, bis = carry
                ibc = jnp.zeros((_LANES,), jnp.int32) + i
                ms2, bis2 = [], []
                for k in range(_NSTREAM):
                    v = buf[pl.ds((k * _SEGV + i) * _LANES, _LANES)]
                    gt = v > ms[k]
                    ms2.append(jnp.where(gt, v, ms[k]))
                    bis2.append(jnp.where(gt, ibc, bis[k]))
                return (tuple(ms2), tuple(bis2))

            minit = tuple(
                jnp.full((_LANES,), -jnp.inf, jnp.float32)
                for _ in range(_NSTREAM))
            binit = tuple(
                jnp.zeros((_LANES,), jnp.int32) for _ in range(_NSTREAM))
            ms, bis = lax.fori_loop(0, _SEGV, step, (minit, binit),
                                    unroll=_UNROLL)
            for k in range(_NSTREAM):
                base = chunk_off + k * _SEGV * _LANES
                idx_k = bis[k] * _LANES + (lanes + base)
                m, bi = _merge(m, bi, ms[k], idx_k)
            return m, bi

        def scan_tail(r, m, bi):
            def step(i, carry):
                m, bi, idx = carry
                v = tailbuf[pl.ds((r * _TAILN // _LANES + i) * _LANES,
                                  _LANES)]
                gt = v > m
                m = jnp.where(gt, v, m)
                bi = jnp.where(gt, idx, bi)
                return (m, bi, idx + _LANES)

            nvec = _TAILN // _LANES
            m, bi, _ = lax.fori_loop(0, nvec, step,
                                     (m, bi, lanes + _TAIL0), unroll=nvec)
            return m, bi

        tail_handles = []
        for r in range(_ROWS_PER_W):
            row = wid * _ROWS_PER_W + r
            tail_handles.append(pltpu.async_copy(
                tail_hbm.at[row], tailbuf.at[pl.ds(r * _TAILN, _TAILN)],
                sem2))

        handles = [None] * nt
        handles[0] = start(0)
        for r in range(_ROWS_PER_W):
            m = jnp.full((_LANES,), -jnp.inf, jnp.float32)
            bi = jnp.zeros((_LANES,), jnp.int32)
            for c in range(_CPR):
                t = r * _CPR + c
                if t + 1 < nt:
                    handles[t + 1] = start(t + 1)
                handles[t].wait()
                m, bi = scan_chunk(bufs[t % 2], _CHUNKS[c][0], m, bi)

            tail_handles[r].wait()
            m, bi = scan_tail(r, m, bi)

            k_sorted, _ = plsc.sort_key_val(m, bi, descending=True)
            rowmax = k_sorted[0]
            cand = jnp.where(m == rowmax, bi, _IMAX)
            c_sorted, _ = plsc.sort_key_val(cand, cand)
            rowidx = c_sorted[0]
            res_ref[...] = jnp.where(lanes == r, rowidx, res_ref[...])

        pltpu.sync_copy(res_ref, out_hbm.at[wid])

    return body


_SC_ARGMAX = _make_sc_argmax()

_TC_BLOCK = 8


def _tc_body(x_ref, o_ref):
    x = x_ref[...]
    m = jnp.max(x, axis=1)
    idxs = lax.broadcasted_iota(jnp.int32, (_TC_BLOCK, _VOCAB), 1)
    cand = jnp.where(x == m[:, None], idxs, _IMAX)
    o_ref[...] = jnp.min(cand, axis=1)[:, None]


_TC_ARGMAX = pl.pallas_call(
    _tc_body,
    grid=(_TC_ROWS // _TC_BLOCK,),
    in_specs=[pl.BlockSpec((_TC_BLOCK, _VOCAB),
                           lambda i: (i + _SC_ROWS // _TC_BLOCK, 0))],
    out_specs=pl.BlockSpec((_TC_BLOCK, 1), lambda i: (i, 0)),
    out_shape=jax.ShapeDtypeStruct((_TC_ROWS, 1), jnp.int32),
)


def kernel(logits, temperature, top_p, top_k):
    del temperature, top_p, top_k
    sc_in = lax.slice(logits, (0, 0), (_SC_ROWS, _VOCAB))
    tail = lax.slice(logits, (0, _TAIL0), (_SC_ROWS, _VOCAB))
    tail = jnp.concatenate(
        [tail, jnp.full((_SC_ROWS, _TAILN - (_VOCAB - _TAIL0)), -jnp.inf,
                        jnp.float32)], axis=1)
    out_sc = _SC_ARGMAX(sc_in, tail)
    out_tc = _TC_ARGMAX(logits)
    sc_idx = out_sc[:, :_ROWS_PER_W].reshape(_SC_ROWS, 1)
    return jnp.concatenate([sc_idx, out_tc], axis=0)
